# Initial kernel scaffold; baseline (speedup 1.0000x reference)
import jax
import jax.numpy as jnp
from jax.experimental import pallas as pl


def kernel(input_ids, position_ids, token_type_ids, W_word, W_pos, W_tt, gamma, beta):
    raise NotImplementedError("write your pallas kernel here")


# speedup vs baseline: 1.6149x; 1.6149x over previous
import functools

import jax
import jax.numpy as jnp
from jax import lax
from jax.experimental import pallas as pl
from jax.experimental.pallas import tpu as pltpu
from jax.experimental.pallas import tpu_sc as plsc

VOCAB = 100000
HIDDEN = 128
MAX_POS = 512
TYPE_VOCAB = 2
EPS = 1e-12
B, T = 1024, 200
N = B * T

NC, NS = 2, 16
NW = NC * NS
TPW = N // NW
C = 128
G = TPW // C
POS_STAGE = 256
L = 16
NV = HIDDEN // L


def _sc_body(w_word, w_pos, w_tt, gamma, beta, wids, pids, tids, out_hbm,
             pos_v, tt_v, gam_v, bet_v, widx, pidx, tidx,
             in0, in1, ot0, ot1, gsem0, gsem1, osem0, osem1):
    cid = lax.axis_index("c")
    sid = lax.axis_index("s")
    wid = sid * NC + cid

    pltpu.sync_copy(w_pos.at[pl.ds(0, POS_STAGE)], pos_v)
    pltpu.sync_copy(w_tt, tt_v)
    pltpu.sync_copy(gamma, gam_v)
    pltpu.sync_copy(beta, bet_v)
    pltpu.sync_copy(wids.at[wid], widx)
    pltpu.sync_copy(pids.at[wid], pidx)
    pltpu.sync_copy(tids.at[wid], tidx)

    ins = (in0, in1)
    ots = (ot0, ot1)
    gsems = (gsem0, gsem1)
    osems = (osem0, osem1)

    def start_gather(g, buf, sem):
        pltpu.async_copy(w_word.at[widx.at[g]], buf, sem)

    def wait_gather(g, buf, sem):
        pltpu.make_async_copy(w_word.at[widx.at[g]], buf, sem).wait()

    def out_slice(g):
        return out_hbm.at[pl.ds((wid * G + g) * C, C)]

    start_gather(0, ins[0], gsems[0])
    start_gather(1, ins[1], gsems[1])

    perms = [jnp.arange(L, dtype=jnp.int32) ^ sh for sh in (8, 4, 2, 1)]

    def lane_sum(v):
        for idx in perms:
            v = v + jnp.take_along_axis(v, idx, axis=0)
        return v

    half = jnp.full((L,), 0.5, jnp.float32)
    three_half = jnp.full((L,), 1.5, jnp.float32)
    magic = jnp.full((L,), 0x5F3759DF, jnp.int32)
    inv_h = jnp.full((L,), 1.0 / HIDDEN, jnp.float32)
    eps_v = jnp.full((L,), EPS, jnp.float32)

    def chunk(g, b):
        inb, otb = ins[b], ots[b]
        wait_gather(g, inb, gsems[b])

        @pl.when(g >= 2)
        def _():
            pltpu.make_async_copy(otb, out_slice(g), osems[b]).wait()

        def group(j, carry):
            pid_v = pidx[g, pl.ds(j * L, L)]
            tid_v = tidx[g, pl.ds(j * L, L)]
            for k in range(L):
                t = j * L + k
                pid = pid_v[k]
                tid = tid_v[k]
                rows = []
                s = jnp.zeros((L,), jnp.float32)
                q = jnp.zeros((L,), jnp.float32)
                for i in range(NV):
                    sl = pl.ds(i * L, L)
                    r = inb[t, sl] + pos_v[pid, sl] + tt_v[tid, sl]
                    rows.append(r)
                    s = s + r
                    q = q + r * r
                mean = lane_sum(s) * inv_h
                var = lane_sum(q) * inv_h - mean * mean
                x = var + eps_v
                y = lax.bitcast_convert_type(
                    magic - lax.shift_right_logical(
                        lax.bitcast_convert_type(x, jnp.int32),
                        jnp.full((L,), 1, jnp.int32)),
                    jnp.float32)
                hx = half * x
                for _ in range(3):
                    y = y * (three_half - hx * y * y)
                for i in range(NV):
                    sl = pl.ds(i * L, L)
                    otb[t, sl] = (rows[i] - mean) * y * gam_v[sl] + bet_v[sl]
            return carry

        lax.fori_loop(0, C // L, group, 0)

        pltpu.async_copy(otb, out_slice(g), osems[b])

        @pl.when(g + 2 < G)
        def _():
            start_gather(g + 2, inb, gsems[b])

    def pair(m, carry):
        chunk(2 * m, 0)
        chunk(2 * m + 1, 1)
        return carry

    lax.fori_loop(0, G // 2, pair, 0)

    pltpu.make_async_copy(ots[0], out_slice(G - 2), osems[0]).wait()
    pltpu.make_async_copy(ots[1], out_slice(G - 1), osems[1]).wait()


@functools.partial(
    pl.kernel,
    out_type=jax.ShapeDtypeStruct((N, HIDDEN), jnp.float32),
    mesh=plsc.VectorSubcoreMesh(core_axis_name="c", subcore_axis_name="s"),
    scratch_types=[
        pltpu.VMEM((POS_STAGE, HIDDEN), jnp.float32),
        pltpu.VMEM((TYPE_VOCAB, HIDDEN), jnp.float32),
        pltpu.VMEM((HIDDEN,), jnp.float32),
        pltpu.VMEM((HIDDEN,), jnp.float32),
        pltpu.VMEM((G, C), jnp.int32),
        pltpu.VMEM((G, C), jnp.int32),
        pltpu.VMEM((G, C), jnp.int32),
        pltpu.VMEM((C, HIDDEN), jnp.float32),
        pltpu.VMEM((C, HIDDEN), jnp.float32),
        pltpu.VMEM((C, HIDDEN), jnp.float32),
        pltpu.VMEM((C, HIDDEN), jnp.float32),
        pltpu.SemaphoreType.DMA,
        pltpu.SemaphoreType.DMA,
        pltpu.SemaphoreType.DMA,
        pltpu.SemaphoreType.DMA,
    ],
)
def _bert_embed_sc(w_word, w_pos, w_tt, gamma, beta, wids, pids, tids, out_hbm,
                   *scratch):
    _sc_body(w_word, w_pos, w_tt, gamma, beta, wids, pids, tids, out_hbm,
             *scratch)


@jax.jit
def kernel(input_ids, position_ids, token_type_ids, W_word, W_pos, W_tt,
           gamma, beta):
    wids = input_ids.reshape(NW, G, C).astype(jnp.int32)
    pids = position_ids.reshape(NW, G, C).astype(jnp.int32)
    tids = token_type_ids.reshape(NW, G, C).astype(jnp.int32)
    out = _bert_embed_sc(W_word, W_pos, W_tt, gamma, beta, wids, pids, tids)
    return out.reshape(B, T, HIDDEN)
